# Initial kernel scaffold; baseline (speedup 1.0000x reference)
import jax
import jax.numpy as jnp
from jax.experimental import pallas as pl


def kernel(x, edge_index, batch_size, pred_steps, W1, b1, W2, b2):
    raise NotImplementedError("write your pallas kernel here")


# speedup vs baseline: 4.5098x; 4.5098x over previous
import functools
import jax
import jax.numpy as jnp
from jax import lax
from jax.experimental import pallas as pl
from jax.experimental.pallas import tpu as pltpu
from jax.experimental.pallas import tpu_sc as plsc

N = 10000
NPAD = 10240
E = 320000
D_IN = 128
D_HID = 256
D_OUT = 16
NW = 32
EW = E // NW
E4 = E * 4
EB = 8000
NF = NPAD * 4
BLK = 128

_SC_PARAMS = pltpu.CompilerParams(needs_layout_passes=False)


def _mesh():
    return plsc.VectorSubcoreMesh(core_axis_name="c", subcore_axis_name="s")


def _zero_ref(ref, nvec):
    def z(i, carry):
        ref[pl.ds(i * 16, 16)] = jnp.zeros((16,), jnp.float32)
        return carry
    lax.fori_loop(0, nvec, z, 0)


def _deg_body(dst, out, idx_v, hist_v):
    c = lax.axis_index("c")
    s = lax.axis_index("s")
    w = c * 16 + s
    pltpu.sync_copy(dst.at[pl.ds(w * EW, EW)], idx_v)
    _zero_ref(hist_v, NPAD // 16)

    def body(j, carry):
        iv = idx_v[pl.ds(j * 16, 16)]
        plsc.addupdate_scatter(hist_v, [iv], jnp.ones((16,), jnp.float32))
        return carry

    lax.fori_loop(0, EW // 16, body, 0)
    pltpu.sync_copy(hist_v, out.at[w])


_deg_call = functools.partial(
    pl.kernel,
    out_type=jax.ShapeDtypeStruct((NW, NPAD), jnp.float32),
    mesh=_mesh(),
    scratch_types=[
        pltpu.VMEM((EW,), jnp.int32),
        pltpu.VMEM((NPAD,), jnp.float32),
    ],
    compiler_params=_SC_PARAMS,
)(_deg_body)


def _agg1_body(y32, srcx, dstx, out, y_v, acc_v, sx_v, dx_v):
    c = lax.axis_index("c")
    s = lax.axis_index("s")
    w = c * 16 + s
    pltpu.sync_copy(y32.at[w], y_v)
    _zero_ref(acc_v, NF // 16)

    def blk(b, carry):
        pltpu.sync_copy(srcx.at[pl.ds(b * EB, EB)], sx_v)
        pltpu.sync_copy(dstx.at[pl.ds(b * EB, EB)], dx_v)

        def body(j, carry2):
            gi = sx_v[pl.ds(j * 16, 16)]
            di = dx_v[pl.ds(j * 16, 16)]
            vals = plsc.load_gather(y_v, [gi])
            plsc.addupdate_scatter(acc_v, [di], vals)
            return carry2

        return lax.fori_loop(0, EB // 16, body, carry)

    lax.fori_loop(0, E4 // EB, blk, 0)
    pltpu.sync_copy(acc_v, out.at[w])


_agg1_call = functools.partial(
    pl.kernel,
    out_type=jax.ShapeDtypeStruct((NW, NF), jnp.float32),
    mesh=_mesh(),
    scratch_types=[
        pltpu.VMEM((NF,), jnp.float32),
        pltpu.VMEM((NF,), jnp.float32),
        pltpu.VMEM((EB,), jnp.int32),
        pltpu.VMEM((EB,), jnp.int32),
    ],
    compiler_params=_SC_PARAMS,
)(_agg1_body)


def _agg2_body(y2g, srcx, dstx, out, y_v, acc_v, sx_v, dx_v):
    c = lax.axis_index("c")
    s = lax.axis_index("s")
    w = c * 16 + s
    g = w % 4
    t = w // 4
    e0 = t * (E4 // 8)
    pltpu.sync_copy(y2g.at[g], y_v)
    _zero_ref(acc_v, NF // 16)

    def blk(b, carry):
        pltpu.sync_copy(srcx.at[pl.ds(e0 + b * EB, EB)], sx_v)
        pltpu.sync_copy(dstx.at[pl.ds(e0 + b * EB, EB)], dx_v)

        def body(j, carry2):
            gi = sx_v[pl.ds(j * 16, 16)]
            di = dx_v[pl.ds(j * 16, 16)]
            vals = plsc.load_gather(y_v, [gi])
            plsc.addupdate_scatter(acc_v, [di], vals)
            return carry2

        return lax.fori_loop(0, EB // 16, body, carry)

    lax.fori_loop(0, E4 // 8 // EB, blk, 0)
    pltpu.sync_copy(acc_v, out.at[w])


_agg2_call = functools.partial(
    pl.kernel,
    out_type=jax.ShapeDtypeStruct((NW, NF), jnp.float32),
    mesh=_mesh(),
    scratch_types=[
        pltpu.VMEM((NF,), jnp.float32),
        pltpu.VMEM((NF,), jnp.float32),
        pltpu.VMEM((EB,), jnp.int32),
        pltpu.VMEM((EB,), jnp.int32),
    ],
    compiler_params=_SC_PARAMS,
)(_agg2_body)


def _prep_body(x_ref, deg_ref, y_ref, dinv_ref):
    deg = jnp.sum(deg_ref[...], axis=0, keepdims=True).T + 1.0
    dinv = lax.rsqrt(deg)
    y_ref[...] = x_ref[...] * dinv
    dinv_ref[...] = jnp.broadcast_to(dinv, (BLK, 16))


def _prep_call(x_pad, deg32):
    return pl.pallas_call(
        _prep_body,
        grid=(NPAD // BLK,),
        in_specs=[
            pl.BlockSpec((BLK, D_IN), lambda i: (i, 0)),
            pl.BlockSpec((NW, BLK), lambda i: (0, i)),
        ],
        out_specs=[
            pl.BlockSpec((BLK, D_IN), lambda i: (i, 0)),
            pl.BlockSpec((BLK, 16), lambda i: (i, 0)),
        ],
        out_shape=[
            jax.ShapeDtypeStruct((NPAD, D_IN), jnp.float32),
            jax.ShapeDtypeStruct((NPAD, 16), jnp.float32),
        ],
    )(x_pad, deg32)


def _mlp_body(acc_ref, y_ref, dinv_ref, w1_ref, b1_ref, w2_ref, y2_ref):
    dinv = dinv_ref[:, 0:1]
    a = (acc_ref[...] + y_ref[...]) * dinv
    h = jnp.dot(a, w1_ref[...], preferred_element_type=jnp.float32)
    h = jnp.maximum(h + b1_ref[...], 0.0)
    h2 = jnp.dot(h, w2_ref[...], preferred_element_type=jnp.float32)
    y2_ref[...] = h2 * dinv


def _mlp_call(acc, y, dinv16, W1, b1, W2):
    return pl.pallas_call(
        _mlp_body,
        grid=(NPAD // BLK,),
        in_specs=[
            pl.BlockSpec((BLK, D_IN), lambda i: (i, 0)),
            pl.BlockSpec((BLK, D_IN), lambda i: (i, 0)),
            pl.BlockSpec((BLK, 16), lambda i: (i, 0)),
            pl.BlockSpec((D_IN, D_HID), lambda i: (0, 0)),
            pl.BlockSpec((1, D_HID), lambda i: (0, 0)),
            pl.BlockSpec((D_HID, D_OUT), lambda i: (0, 0)),
        ],
        out_specs=pl.BlockSpec((BLK, D_OUT), lambda i: (i, 0)),
        out_shape=jax.ShapeDtypeStruct((NPAD, D_OUT), jnp.float32),
    )(acc, y, dinv16, W1, b1, W2)


def _final_body(parts_ref, y2_ref, dinv_ref, b2_ref, out_ref):
    dinv = dinv_ref[:, 0:1]
    agg = jnp.sum(parts_ref[...], axis=0)
    v = (agg + y2_ref[...]) * dinv + b2_ref[...]
    m = jnp.max(v, axis=1, keepdims=True)
    lse = jnp.log(jnp.sum(jnp.exp(v - m), axis=1, keepdims=True)) + m
    out_ref[...] = v - lse


def _final_call(acc2_parts, y2, dinv16, b2):
    return pl.pallas_call(
        _final_body,
        grid=(NPAD // BLK,),
        in_specs=[
            pl.BlockSpec((8, BLK, D_OUT), lambda i: (0, i, 0)),
            pl.BlockSpec((BLK, D_OUT), lambda i: (i, 0)),
            pl.BlockSpec((BLK, 16), lambda i: (i, 0)),
            pl.BlockSpec((1, D_OUT), lambda i: (0, 0)),
        ],
        out_specs=pl.BlockSpec((BLK, D_OUT), lambda i: (i, 0)),
        out_shape=jax.ShapeDtypeStruct((NPAD, D_OUT), jnp.float32),
    )(acc2_parts, y2, dinv16, b2)


def kernel(x, edge_index, batch_size, pred_steps, W1, b1, W2, b2):
    x_pad = jnp.pad(x, ((0, NPAD - N), (0, 0)))
    src = edge_index[0]
    dst = edge_index[1]
    lane4 = jnp.arange(4, dtype=jnp.int32)
    srcx = (src[:, None] * 4 + lane4[None, :]).reshape(E4)
    dstx = (dst[:, None] * 4 + lane4[None, :]).reshape(E4)

    deg32 = _deg_call(dst)
    y, dinv16 = _prep_call(x_pad, deg32)

    y32 = jnp.transpose(y.reshape(NPAD, NW, 4), (1, 0, 2)).reshape(NW, NF)
    acc32 = _agg1_call(y32, srcx, dstx)
    acc = jnp.transpose(acc32.reshape(NW, NPAD, 4), (1, 0, 2)).reshape(NPAD, D_IN)

    y2 = _mlp_call(acc, y, dinv16, W1, b1.reshape(1, D_HID), W2)

    y2g = jnp.transpose(y2.reshape(NPAD, 4, 4), (1, 0, 2)).reshape(4, NF)
    acc2_32 = _agg2_call(y2g, srcx, dstx)
    acc2_parts = jnp.transpose(
        acc2_32.reshape(8, 4, NPAD, 4), (0, 2, 1, 3)).reshape(8, NPAD, D_OUT)

    res = _final_call(acc2_parts, y2, dinv16, b2.reshape(1, D_OUT))

    return jnp.broadcast_to(res[:N][None, None], (4, 3, N, D_OUT))


# speedup vs baseline: 4.6718x; 1.0359x over previous
import functools
import jax
import jax.numpy as jnp
from jax import lax
from jax.experimental import pallas as pl
from jax.experimental.pallas import tpu as pltpu
from jax.experimental.pallas import tpu_sc as plsc

N = 10000
NPAD = 10240
E = 320000
D_IN = 128
D_HID = 256
D_OUT = 16
NW = 32
EW = E // NW
E4 = E * 4
EB = 8000
NF = NPAD * 4
BLK = 128

_SC_PARAMS = pltpu.CompilerParams(needs_layout_passes=False)


def _mesh():
    return plsc.VectorSubcoreMesh(core_axis_name="c", subcore_axis_name="s")


def _zero_ref(ref, nvec):
    def z(i, carry):
        ref[pl.ds(i * 16, 16)] = jnp.zeros((16,), jnp.float32)
        return carry
    lax.fori_loop(0, nvec, z, 0)


def _deg_body(dst, out, idx_v, hist_v):
    c = lax.axis_index("c")
    s = lax.axis_index("s")
    w = c * 16 + s
    pltpu.sync_copy(dst.at[pl.ds(w * EW, EW)], idx_v)
    _zero_ref(hist_v, NPAD // 16)

    def body(j, carry):
        iv = idx_v[pl.ds(j * 16, 16)]
        plsc.addupdate_scatter(hist_v, [iv], jnp.ones((16,), jnp.float32))
        return carry

    lax.fori_loop(0, EW // 16, body, 0, unroll=8)
    pltpu.sync_copy(hist_v, out.at[w])


_deg_call = functools.partial(
    pl.kernel,
    out_type=jax.ShapeDtypeStruct((NW, NPAD), jnp.float32),
    mesh=_mesh(),
    scratch_types=[
        pltpu.VMEM((EW,), jnp.int32),
        pltpu.VMEM((NPAD,), jnp.float32),
    ],
    compiler_params=_SC_PARAMS,
)(_deg_body)


def _agg1_body(y32, srcx, dstx, out, y_v, acc_v, sx_v, dx_v):
    c = lax.axis_index("c")
    s = lax.axis_index("s")
    w = c * 16 + s
    pltpu.sync_copy(y32.at[w], y_v)
    _zero_ref(acc_v, NF // 16)

    def blk(b, carry):
        pltpu.sync_copy(srcx.at[pl.ds(b * EB, EB)], sx_v)
        pltpu.sync_copy(dstx.at[pl.ds(b * EB, EB)], dx_v)

        def body(j, carry2):
            gi = sx_v[pl.ds(j * 16, 16)]
            di = dx_v[pl.ds(j * 16, 16)]
            vals = plsc.load_gather(y_v, [gi])
            plsc.addupdate_scatter(acc_v, [di], vals)
            return carry2

        return lax.fori_loop(0, EB // 16, body, carry, unroll=8)

    lax.fori_loop(0, E4 // EB, blk, 0)
    pltpu.sync_copy(acc_v, out.at[w])


_agg1_call = functools.partial(
    pl.kernel,
    out_type=jax.ShapeDtypeStruct((NW, NF), jnp.float32),
    mesh=_mesh(),
    scratch_types=[
        pltpu.VMEM((NF,), jnp.float32),
        pltpu.VMEM((NF,), jnp.float32),
        pltpu.VMEM((EB,), jnp.int32),
        pltpu.VMEM((EB,), jnp.int32),
    ],
    compiler_params=_SC_PARAMS,
)(_agg1_body)


def _agg2_body(y2g, srcx, dstx, out, y_v, acc_v, sx_v, dx_v):
    c = lax.axis_index("c")
    s = lax.axis_index("s")
    w = c * 16 + s
    g = w % 4
    t = w // 4
    e0 = t * (E4 // 8)
    pltpu.sync_copy(y2g.at[g], y_v)
    _zero_ref(acc_v, NF // 16)

    def blk(b, carry):
        pltpu.sync_copy(srcx.at[pl.ds(e0 + b * EB, EB)], sx_v)
        pltpu.sync_copy(dstx.at[pl.ds(e0 + b * EB, EB)], dx_v)

        def body(j, carry2):
            gi = sx_v[pl.ds(j * 16, 16)]
            di = dx_v[pl.ds(j * 16, 16)]
            vals = plsc.load_gather(y_v, [gi])
            plsc.addupdate_scatter(acc_v, [di], vals)
            return carry2

        return lax.fori_loop(0, EB // 16, body, carry, unroll=8)

    lax.fori_loop(0, E4 // 8 // EB, blk, 0)
    pltpu.sync_copy(acc_v, out.at[w])


_agg2_call = functools.partial(
    pl.kernel,
    out_type=jax.ShapeDtypeStruct((NW, NF), jnp.float32),
    mesh=_mesh(),
    scratch_types=[
        pltpu.VMEM((NF,), jnp.float32),
        pltpu.VMEM((NF,), jnp.float32),
        pltpu.VMEM((EB,), jnp.int32),
        pltpu.VMEM((EB,), jnp.int32),
    ],
    compiler_params=_SC_PARAMS,
)(_agg2_body)


def _prep_body(x_ref, deg_ref, y_ref, dinv_ref):
    deg = jnp.sum(deg_ref[...], axis=0, keepdims=True).T + 1.0
    dinv = lax.rsqrt(deg)
    y_ref[...] = x_ref[...] * dinv
    dinv_ref[...] = jnp.broadcast_to(dinv, (BLK, 16))


def _prep_call(x_pad, deg32):
    return pl.pallas_call(
        _prep_body,
        grid=(NPAD // BLK,),
        in_specs=[
            pl.BlockSpec((BLK, D_IN), lambda i: (i, 0)),
            pl.BlockSpec((NW, BLK), lambda i: (0, i)),
        ],
        out_specs=[
            pl.BlockSpec((BLK, D_IN), lambda i: (i, 0)),
            pl.BlockSpec((BLK, 16), lambda i: (i, 0)),
        ],
        out_shape=[
            jax.ShapeDtypeStruct((NPAD, D_IN), jnp.float32),
            jax.ShapeDtypeStruct((NPAD, 16), jnp.float32),
        ],
    )(x_pad, deg32)


def _mlp_body(acc_ref, y_ref, dinv_ref, w1_ref, b1_ref, w2_ref, y2_ref):
    dinv = dinv_ref[:, 0:1]
    a = (acc_ref[...] + y_ref[...]) * dinv
    h = jnp.dot(a, w1_ref[...], preferred_element_type=jnp.float32)
    h = jnp.maximum(h + b1_ref[...], 0.0)
    h2 = jnp.dot(h, w2_ref[...], preferred_element_type=jnp.float32)
    y2_ref[...] = h2 * dinv


def _mlp_call(acc, y, dinv16, W1, b1, W2):
    return pl.pallas_call(
        _mlp_body,
        grid=(NPAD // BLK,),
        in_specs=[
            pl.BlockSpec((BLK, D_IN), lambda i: (i, 0)),
            pl.BlockSpec((BLK, D_IN), lambda i: (i, 0)),
            pl.BlockSpec((BLK, 16), lambda i: (i, 0)),
            pl.BlockSpec((D_IN, D_HID), lambda i: (0, 0)),
            pl.BlockSpec((1, D_HID), lambda i: (0, 0)),
            pl.BlockSpec((D_HID, D_OUT), lambda i: (0, 0)),
        ],
        out_specs=pl.BlockSpec((BLK, D_OUT), lambda i: (i, 0)),
        out_shape=jax.ShapeDtypeStruct((NPAD, D_OUT), jnp.float32),
    )(acc, y, dinv16, W1, b1, W2)


def _final_body(parts_ref, y2_ref, dinv_ref, b2_ref, out_ref):
    dinv = dinv_ref[:, 0:1]
    agg = jnp.sum(parts_ref[...], axis=0)
    v = (agg + y2_ref[...]) * dinv + b2_ref[...]
    m = jnp.max(v, axis=1, keepdims=True)
    lse = jnp.log(jnp.sum(jnp.exp(v - m), axis=1, keepdims=True)) + m
    out_ref[...] = v - lse


def _final_call(acc2_parts, y2, dinv16, b2):
    return pl.pallas_call(
        _final_body,
        grid=(NPAD // BLK,),
        in_specs=[
            pl.BlockSpec((8, BLK, D_OUT), lambda i: (0, i, 0)),
            pl.BlockSpec((BLK, D_OUT), lambda i: (i, 0)),
            pl.BlockSpec((BLK, 16), lambda i: (i, 0)),
            pl.BlockSpec((1, D_OUT), lambda i: (0, 0)),
        ],
        out_specs=pl.BlockSpec((BLK, D_OUT), lambda i: (i, 0)),
        out_shape=jax.ShapeDtypeStruct((NPAD, D_OUT), jnp.float32),
    )(acc2_parts, y2, dinv16, b2)


def kernel(x, edge_index, batch_size, pred_steps, W1, b1, W2, b2):
    x_pad = jnp.pad(x, ((0, NPAD - N), (0, 0)))
    src = edge_index[0]
    dst = edge_index[1]
    lane4 = jnp.arange(4, dtype=jnp.int32)
    srcx = (src[:, None] * 4 + lane4[None, :]).reshape(E4)
    dstx = (dst[:, None] * 4 + lane4[None, :]).reshape(E4)

    deg32 = _deg_call(dst)
    y, dinv16 = _prep_call(x_pad, deg32)

    y32 = jnp.transpose(y.reshape(NPAD, NW, 4), (1, 0, 2)).reshape(NW, NF)
    acc32 = _agg1_call(y32, srcx, dstx)
    acc = jnp.transpose(acc32.reshape(NW, NPAD, 4), (1, 0, 2)).reshape(NPAD, D_IN)

    y2 = _mlp_call(acc, y, dinv16, W1, b1.reshape(1, D_HID), W2)

    y2g = jnp.transpose(y2.reshape(NPAD, 4, 4), (1, 0, 2)).reshape(4, NF)
    acc2_32 = _agg2_call(y2g, srcx, dstx)
    acc2_parts = jnp.transpose(
        acc2_32.reshape(8, 4, NPAD, 4), (0, 2, 1, 3)).reshape(8, NPAD, D_OUT)

    res = _final_call(acc2_parts, y2, dinv16, b2.reshape(1, D_OUT))

    return jnp.broadcast_to(res[:N][None, None], (4, 3, N, D_OUT))


# speedup vs baseline: 6.8473x; 1.4657x over previous
import functools
import jax
import jax.numpy as jnp
from jax import lax
from jax.experimental import pallas as pl
from jax.experimental.pallas import tpu as pltpu
from jax.experimental.pallas import tpu_sc as plsc

N = 10000
NPAD = 10240
E = 320000
D_IN = 128
D_HID = 256
D_OUT = 16
NW = 32
EW = E // NW
EB = 4000
NF = NPAD * 4
BLK = 128

_SC_PARAMS = pltpu.CompilerParams(needs_layout_passes=False)


def _mesh():
    return plsc.VectorSubcoreMesh(core_axis_name="c", subcore_axis_name="s")


def _zero_ref(ref, nvec):
    def z(i, carry):
        ref[pl.ds(i * 16, 16)] = jnp.zeros((16,), jnp.float32)
        return carry
    lax.fori_loop(0, nvec, z, 0)


def _deg_body(dst, out, idx_v, hist_v):
    c = lax.axis_index("c")
    s = lax.axis_index("s")
    w = c * 16 + s
    pltpu.sync_copy(dst.at[pl.ds(w * EW, EW)], idx_v)
    _zero_ref(hist_v, NPAD // 16)

    def body(j, carry):
        iv = idx_v[pl.ds(j * 16, 16)]
        plsc.addupdate_scatter(hist_v, [iv], jnp.ones((16,), jnp.float32))
        return carry

    lax.fori_loop(0, EW // 16, body, 0, unroll=8)
    pltpu.sync_copy(hist_v, out.at[w])


_deg_call = functools.partial(
    pl.kernel,
    out_type=jax.ShapeDtypeStruct((NW, NPAD), jnp.float32),
    mesh=_mesh(),
    scratch_types=[
        pltpu.VMEM((EW,), jnp.int32),
        pltpu.VMEM((NPAD,), jnp.float32),
    ],
    compiler_params=_SC_PARAMS,
)(_deg_body)


def _agg1_body(y32, src, dst, out, y_v, acc_v, sx_v, dx_v):
    c = lax.axis_index("c")
    s = lax.axis_index("s")
    w = c * 16 + s
    pltpu.sync_copy(y32.at[w], y_v)
    _zero_ref(acc_v, NF // 16)

    def blk(b, carry):
        pltpu.sync_copy(src.at[pl.ds(b * EB, EB)], sx_v)
        pltpu.sync_copy(dst.at[pl.ds(b * EB, EB)], dx_v)

        def body(j, carry2):
            sv = sx_v[pl.ds(j * 16, 16)] * 4
            dv = dx_v[pl.ds(j * 16, 16)] * 4
            for k in range(4):
                vals = plsc.load_gather(y_v, [sv + k])
                plsc.addupdate_scatter(acc_v, [dv + k], vals)
            return carry2

        return lax.fori_loop(0, EB // 16, body, carry, unroll=4)

    lax.fori_loop(0, E // EB, blk, 0)
    pltpu.sync_copy(acc_v, out.at[w])


_agg1_call = functools.partial(
    pl.kernel,
    out_type=jax.ShapeDtypeStruct((NW, NF), jnp.float32),
    mesh=_mesh(),
    scratch_types=[
        pltpu.VMEM((NF,), jnp.float32),
        pltpu.VMEM((NF,), jnp.float32),
        pltpu.VMEM((EB,), jnp.int32),
        pltpu.VMEM((EB,), jnp.int32),
    ],
    compiler_params=_SC_PARAMS,
)(_agg1_body)


def _agg2_body(y2g, src, dst, out, y_v, acc_v, sx_v, dx_v):
    c = lax.axis_index("c")
    s = lax.axis_index("s")
    w = c * 16 + s
    g = w % 4
    t = w // 4
    e0 = t * (E // 8)
    pltpu.sync_copy(y2g.at[g], y_v)
    _zero_ref(acc_v, NF // 16)

    def blk(b, carry):
        pltpu.sync_copy(src.at[pl.ds(e0 + b * EB, EB)], sx_v)
        pltpu.sync_copy(dst.at[pl.ds(e0 + b * EB, EB)], dx_v)

        def body(j, carry2):
            sv = sx_v[pl.ds(j * 16, 16)] * 4
            dv = dx_v[pl.ds(j * 16, 16)] * 4
            for k in range(4):
                vals = plsc.load_gather(y_v, [sv + k])
                plsc.addupdate_scatter(acc_v, [dv + k], vals)
            return carry2

        return lax.fori_loop(0, EB // 16, body, carry, unroll=4)

    lax.fori_loop(0, E // 8 // EB, blk, 0)
    pltpu.sync_copy(acc_v, out.at[w])


_agg2_call = functools.partial(
    pl.kernel,
    out_type=jax.ShapeDtypeStruct((NW, NF), jnp.float32),
    mesh=_mesh(),
    scratch_types=[
        pltpu.VMEM((NF,), jnp.float32),
        pltpu.VMEM((NF,), jnp.float32),
        pltpu.VMEM((EB,), jnp.int32),
        pltpu.VMEM((EB,), jnp.int32),
    ],
    compiler_params=_SC_PARAMS,
)(_agg2_body)


def _prep_body(x_ref, deg_ref, y_ref, dinv_ref):
    deg = jnp.sum(deg_ref[...], axis=0, keepdims=True).T + 1.0
    dinv = lax.rsqrt(deg)
    y_ref[...] = x_ref[...] * dinv
    dinv_ref[...] = jnp.broadcast_to(dinv, (BLK, 16))


def _prep_call(x_pad, deg32):
    return pl.pallas_call(
        _prep_body,
        grid=(NPAD // BLK,),
        in_specs=[
            pl.BlockSpec((BLK, D_IN), lambda i: (i, 0)),
            pl.BlockSpec((NW, BLK), lambda i: (0, i)),
        ],
        out_specs=[
            pl.BlockSpec((BLK, D_IN), lambda i: (i, 0)),
            pl.BlockSpec((BLK, 16), lambda i: (i, 0)),
        ],
        out_shape=[
            jax.ShapeDtypeStruct((NPAD, D_IN), jnp.float32),
            jax.ShapeDtypeStruct((NPAD, 16), jnp.float32),
        ],
    )(x_pad, deg32)


def _mlp_body(acc_ref, y_ref, dinv_ref, w1_ref, b1_ref, w2_ref, y2_ref):
    dinv = dinv_ref[:, 0:1]
    a = (acc_ref[...] + y_ref[...]) * dinv
    h = jnp.dot(a, w1_ref[...], preferred_element_type=jnp.float32)
    h = jnp.maximum(h + b1_ref[...], 0.0)
    h2 = jnp.dot(h, w2_ref[...], preferred_element_type=jnp.float32)
    y2_ref[...] = h2 * dinv


def _mlp_call(acc, y, dinv16, W1, b1, W2):
    return pl.pallas_call(
        _mlp_body,
        grid=(NPAD // BLK,),
        in_specs=[
            pl.BlockSpec((BLK, D_IN), lambda i: (i, 0)),
            pl.BlockSpec((BLK, D_IN), lambda i: (i, 0)),
            pl.BlockSpec((BLK, 16), lambda i: (i, 0)),
            pl.BlockSpec((D_IN, D_HID), lambda i: (0, 0)),
            pl.BlockSpec((1, D_HID), lambda i: (0, 0)),
            pl.BlockSpec((D_HID, D_OUT), lambda i: (0, 0)),
        ],
        out_specs=pl.BlockSpec((BLK, D_OUT), lambda i: (i, 0)),
        out_shape=jax.ShapeDtypeStruct((NPAD, D_OUT), jnp.float32),
    )(acc, y, dinv16, W1, b1, W2)


def _final_body(parts_ref, y2_ref, dinv_ref, b2_ref, out_ref):
    dinv = dinv_ref[:, 0:1]
    agg = jnp.sum(parts_ref[...], axis=0)
    v = (agg + y2_ref[...]) * dinv + b2_ref[...]
    m = jnp.max(v, axis=1, keepdims=True)
    lse = jnp.log(jnp.sum(jnp.exp(v - m), axis=1, keepdims=True)) + m
    out_ref[...] = v - lse


def _final_call(acc2_parts, y2, dinv16, b2):
    return pl.pallas_call(
        _final_body,
        grid=(NPAD // BLK,),
        in_specs=[
            pl.BlockSpec((8, BLK, D_OUT), lambda i: (0, i, 0)),
            pl.BlockSpec((BLK, D_OUT), lambda i: (i, 0)),
            pl.BlockSpec((BLK, 16), lambda i: (i, 0)),
            pl.BlockSpec((1, D_OUT), lambda i: (0, 0)),
        ],
        out_specs=pl.BlockSpec((BLK, D_OUT), lambda i: (i, 0)),
        out_shape=jax.ShapeDtypeStruct((NPAD, D_OUT), jnp.float32),
    )(acc2_parts, y2, dinv16, b2)


def kernel(x, edge_index, batch_size, pred_steps, W1, b1, W2, b2):
    x_pad = jnp.pad(x, ((0, NPAD - N), (0, 0)))
    src = edge_index[0]
    dst = edge_index[1]

    deg32 = _deg_call(dst)
    y, dinv16 = _prep_call(x_pad, deg32)

    y32 = jnp.transpose(y.reshape(NPAD, NW, 4), (1, 0, 2)).reshape(NW, NF)
    acc32 = _agg1_call(y32, src, dst)
    acc = jnp.transpose(acc32.reshape(NW, NPAD, 4), (1, 0, 2)).reshape(NPAD, D_IN)

    y2 = _mlp_call(acc, y, dinv16, W1, b1.reshape(1, D_HID), W2)

    y2g = jnp.transpose(y2.reshape(NPAD, 4, 4), (1, 0, 2)).reshape(4, NF)
    acc2_32 = _agg2_call(y2g, src, dst)
    acc2_parts = jnp.transpose(
        acc2_32.reshape(8, 4, NPAD, 4), (0, 2, 1, 3)).reshape(8, NPAD, D_OUT)

    res = _final_call(acc2_parts, y2, dinv16, b2.reshape(1, D_OUT))

    return jnp.broadcast_to(res[:N][None, None], (4, 3, N, D_OUT))


# speedup vs baseline: 7.0737x; 1.0331x over previous
import functools
import jax
import jax.numpy as jnp
from jax import lax
from jax.experimental import pallas as pl
from jax.experimental.pallas import tpu as pltpu
from jax.experimental.pallas import tpu_sc as plsc

N = 10000
NPAD = 10240
E = 320000
D_IN = 128
D_HID = 256
D_OUT = 16
NW = 32
EW = E // NW
EB = 8000
NF = NPAD * 4
BLK = 128

_SC_PARAMS = pltpu.CompilerParams(needs_layout_passes=False)


def _mesh():
    return plsc.VectorSubcoreMesh(core_axis_name="c", subcore_axis_name="s")


def _zero_ref(ref, nvec):
    def z(i, carry):
        ref[pl.ds(i * 16, 16)] = jnp.zeros((16,), jnp.float32)
        return carry
    lax.fori_loop(0, nvec, z, 0)


def _deg_body(dst, out, idx_v, hist_v):
    c = lax.axis_index("c")
    s = lax.axis_index("s")
    w = c * 16 + s
    pltpu.sync_copy(dst.at[pl.ds(w * EW, EW)], idx_v)
    _zero_ref(hist_v, NPAD // 16)

    def body(j, carry):
        iv = idx_v[pl.ds(j * 16, 16)]
        plsc.addupdate_scatter(hist_v, [iv], jnp.ones((16,), jnp.float32))
        return carry

    lax.fori_loop(0, EW // 16, body, 0, unroll=8)
    pltpu.sync_copy(hist_v, out.at[w])


_deg_call = functools.partial(
    pl.kernel,
    out_type=jax.ShapeDtypeStruct((NW, NPAD), jnp.float32),
    mesh=_mesh(),
    scratch_types=[
        pltpu.VMEM((EW,), jnp.int32),
        pltpu.VMEM((NPAD,), jnp.float32),
    ],
    compiler_params=_SC_PARAMS,
)(_deg_body)


def _agg1_body(y32, src, dst, out, y_v, acc_v, sx_v, dx_v):
    c = lax.axis_index("c")
    s = lax.axis_index("s")
    w = c * 16 + s
    pltpu.sync_copy(y32.at[w], y_v)
    _zero_ref(acc_v, NF // 16)

    def blk(b, carry):
        pltpu.sync_copy(src.at[pl.ds(b * EB, EB)], sx_v)
        pltpu.sync_copy(dst.at[pl.ds(b * EB, EB)], dx_v)

        def body(j, carry2):
            sv = sx_v[pl.ds(j * 16, 16)] * 4
            dv = dx_v[pl.ds(j * 16, 16)] * 4
            for k in range(4):
                vals = plsc.load_gather(y_v, [sv + k])
                plsc.addupdate_scatter(acc_v, [dv + k], vals)
            return carry2

        return lax.fori_loop(0, EB // 16, body, carry, unroll=4)

    lax.fori_loop(0, E // EB, blk, 0)
    pltpu.sync_copy(acc_v, out.at[w])


_agg1_call = functools.partial(
    pl.kernel,
    out_type=jax.ShapeDtypeStruct((NW, NF), jnp.float32),
    mesh=_mesh(),
    scratch_types=[
        pltpu.VMEM((NF,), jnp.float32),
        pltpu.VMEM((NF,), jnp.float32),
        pltpu.VMEM((EB,), jnp.int32),
        pltpu.VMEM((EB,), jnp.int32),
    ],
    compiler_params=_SC_PARAMS,
)(_agg1_body)


def _agg2_body(y2g, src, dst, out, y_v, acc_v, sx_v, dx_v):
    c = lax.axis_index("c")
    s = lax.axis_index("s")
    w = c * 16 + s
    g = w % 4
    t = w // 4
    e0 = t * (E // 8)
    pltpu.sync_copy(y2g.at[g], y_v)
    _zero_ref(acc_v, NF // 16)

    def blk(b, carry):
        pltpu.sync_copy(src.at[pl.ds(e0 + b * EB, EB)], sx_v)
        pltpu.sync_copy(dst.at[pl.ds(e0 + b * EB, EB)], dx_v)

        def body(j, carry2):
            sv = sx_v[pl.ds(j * 16, 16)] * 4
            dv = dx_v[pl.ds(j * 16, 16)] * 4
            for k in range(4):
                vals = plsc.load_gather(y_v, [sv + k])
                plsc.addupdate_scatter(acc_v, [dv + k], vals)
            return carry2

        return lax.fori_loop(0, EB // 16, body, carry, unroll=4)

    lax.fori_loop(0, E // 8 // EB, blk, 0)
    pltpu.sync_copy(acc_v, out.at[w])


_agg2_call = functools.partial(
    pl.kernel,
    out_type=jax.ShapeDtypeStruct((NW, NF), jnp.float32),
    mesh=_mesh(),
    scratch_types=[
        pltpu.VMEM((NF,), jnp.float32),
        pltpu.VMEM((NF,), jnp.float32),
        pltpu.VMEM((EB,), jnp.int32),
        pltpu.VMEM((EB,), jnp.int32),
    ],
    compiler_params=_SC_PARAMS,
)(_agg2_body)


def _prep_body(x_ref, deg_ref, y_ref, dinv_ref):
    deg = jnp.sum(deg_ref[...], axis=0, keepdims=True).T + 1.0
    dinv = lax.rsqrt(deg)
    y_ref[...] = x_ref[...] * dinv
    dinv_ref[...] = jnp.broadcast_to(dinv, (BLK, 16))


def _prep_call(x_pad, deg32):
    return pl.pallas_call(
        _prep_body,
        grid=(NPAD // BLK,),
        in_specs=[
            pl.BlockSpec((BLK, D_IN), lambda i: (i, 0)),
            pl.BlockSpec((NW, BLK), lambda i: (0, i)),
        ],
        out_specs=[
            pl.BlockSpec((BLK, D_IN), lambda i: (i, 0)),
            pl.BlockSpec((BLK, 16), lambda i: (i, 0)),
        ],
        out_shape=[
            jax.ShapeDtypeStruct((NPAD, D_IN), jnp.float32),
            jax.ShapeDtypeStruct((NPAD, 16), jnp.float32),
        ],
    )(x_pad, deg32)


def _mlp_body(acc_ref, y_ref, dinv_ref, w1_ref, b1_ref, w2_ref, y2_ref):
    dinv = dinv_ref[:, 0:1]
    a = (acc_ref[...] + y_ref[...]) * dinv
    h = jnp.dot(a, w1_ref[...], preferred_element_type=jnp.float32)
    h = jnp.maximum(h + b1_ref[...], 0.0)
    h2 = jnp.dot(h, w2_ref[...], preferred_element_type=jnp.float32)
    y2_ref[...] = h2 * dinv


def _mlp_call(acc, y, dinv16, W1, b1, W2):
    return pl.pallas_call(
        _mlp_body,
        grid=(NPAD // BLK,),
        in_specs=[
            pl.BlockSpec((BLK, D_IN), lambda i: (i, 0)),
            pl.BlockSpec((BLK, D_IN), lambda i: (i, 0)),
            pl.BlockSpec((BLK, 16), lambda i: (i, 0)),
            pl.BlockSpec((D_IN, D_HID), lambda i: (0, 0)),
            pl.BlockSpec((1, D_HID), lambda i: (0, 0)),
            pl.BlockSpec((D_HID, D_OUT), lambda i: (0, 0)),
        ],
        out_specs=pl.BlockSpec((BLK, D_OUT), lambda i: (i, 0)),
        out_shape=jax.ShapeDtypeStruct((NPAD, D_OUT), jnp.float32),
    )(acc, y, dinv16, W1, b1, W2)


def _final_body(parts_ref, y2_ref, dinv_ref, b2_ref, out_ref):
    dinv = dinv_ref[:, 0:1]
    agg = jnp.sum(parts_ref[...], axis=0)
    v = (agg + y2_ref[...]) * dinv + b2_ref[...]
    m = jnp.max(v, axis=1, keepdims=True)
    lse = jnp.log(jnp.sum(jnp.exp(v - m), axis=1, keepdims=True)) + m
    out_ref[...] = v - lse


def _final_call(acc2_parts, y2, dinv16, b2):
    return pl.pallas_call(
        _final_body,
        grid=(NPAD // BLK,),
        in_specs=[
            pl.BlockSpec((8, BLK, D_OUT), lambda i: (0, i, 0)),
            pl.BlockSpec((BLK, D_OUT), lambda i: (i, 0)),
            pl.BlockSpec((BLK, 16), lambda i: (i, 0)),
            pl.BlockSpec((1, D_OUT), lambda i: (0, 0)),
        ],
        out_specs=pl.BlockSpec((BLK, D_OUT), lambda i: (i, 0)),
        out_shape=jax.ShapeDtypeStruct((NPAD, D_OUT), jnp.float32),
    )(acc2_parts, y2, dinv16, b2)


def kernel(x, edge_index, batch_size, pred_steps, W1, b1, W2, b2):
    x_pad = jnp.pad(x, ((0, NPAD - N), (0, 0)))
    src = edge_index[0]
    dst = edge_index[1]

    deg32 = _deg_call(dst)
    y, dinv16 = _prep_call(x_pad, deg32)

    y32 = jnp.transpose(y.reshape(NPAD, NW, 4), (1, 0, 2)).reshape(NW, NF)
    acc32 = _agg1_call(y32, src, dst)
    acc = jnp.transpose(acc32.reshape(NW, NPAD, 4), (1, 0, 2)).reshape(NPAD, D_IN)

    y2 = _mlp_call(acc, y, dinv16, W1, b1.reshape(1, D_HID), W2)

    y2g = jnp.transpose(y2.reshape(NPAD, 4, 4), (1, 0, 2)).reshape(4, NF)
    acc2_32 = _agg2_call(y2g, src, dst)
    acc2_parts = jnp.transpose(
        acc2_32.reshape(8, 4, NPAD, 4), (0, 2, 1, 3)).reshape(8, NPAD, D_OUT)

    res = _final_call(acc2_parts, y2, dinv16, b2.reshape(1, D_OUT))

    return jnp.broadcast_to(res[:N][None, None], (4, 3, N, D_OUT))
